# baseline (speedup 1.0000x reference)
import jax
import jax.numpy as jnp
from jax.experimental import pallas as pl

_N_NODES = 10000
_N_EDGES = 160000
_KERNEL = 25
_DIM = 3
_S = 8


def _basis_body(px_ref, py_ref, pz_ref, basis_ref, wi_ref):
    ps = [px_ref[...], py_ref[...], pz_ref[...]]
    v = [p * (_KERNEL - 1) for p in ps]
    lo = [jnp.floor(x) for x in v]
    frac = [x - l for x, l in zip(v, lo)]
    lo_i = [l.astype(jnp.int32) for l in lo]
    for s in range(_S):
        b = jnp.ones_like(ps[0])
        wi = jnp.zeros(ps[0].shape, jnp.int32)
        off = 1
        for d in range(_DIM):
            km = (s >> d) & 1
            b = b * (frac[d] if km == 1 else (1.0 - frac[d]))
            wd = jnp.clip(lo_i[d] + km, 0, _KERNEL - 1)
            wi = wi + wd * off
            off *= _KERNEL
        basis_ref[s, :, :] = b
        wi_ref[s, :, :] = wi


def _spline_basis_pallas(pseudo):
    e = pseudo.shape[0]
    r, c = 1250, 128
    px = pseudo[:, 0].reshape(r, c)
    py = pseudo[:, 1].reshape(r, c)
    pz = pseudo[:, 2].reshape(r, c)
    basis8, wi8 = pl.pallas_call(
        _basis_body,
        out_shape=(
            jax.ShapeDtypeStruct((_S, r, c), jnp.float32),
            jax.ShapeDtypeStruct((_S, r, c), jnp.int32),
        ),
    )(px, py, pz)
    basis = jnp.moveaxis(basis8.reshape(_S, e), 0, 1)
    wi = jnp.moveaxis(wi8.reshape(_S, e), 0, 1)
    return basis, wi


def _spline_message(x_j, basis, wi, W):
    msg = jnp.zeros((x_j.shape[0], W.shape[2]), x_j.dtype)
    for s in range(basis.shape[1]):
        Wg = W[wi[:, s]]
        msg = msg + basis[:, s, None] * jnp.einsum("ei,eio->eo", x_j, Wg)
    return msg


def kernel(x, edge_index, pseudo, params):
    src = edge_index[0]
    dst = edge_index[1]
    basis, wi = _spline_basis_pallas(pseudo)
    h = x
    cnt = jax.ops.segment_sum(jnp.ones((dst.shape[0],), x.dtype), dst, num_segments=_N_NODES)
    inv_cnt = 1.0 / jnp.clip(cnt, 1.0, None)
    for i in range(1, 13):
        W = params[f"W{i}"]
        root = params[f"root{i}"]
        bias = params[f"bias{i}"]
        x_j = h[src]
        msg = _spline_message(x_j, basis, wi, W)
        summed = jax.ops.segment_sum(msg, dst, num_segments=_N_NODES)
        h_new = summed * inv_cnt[:, None] + h @ root + bias
        if i < 12:
            h_new = jax.nn.elu(h_new)
            mu = jnp.mean(h_new, 0)
            var = jnp.var(h_new, 0)
            h_new = (h_new - mu) * jax.lax.rsqrt(var + 1e-5) * params[f"gamma{i}"] + params[f"beta{i}"]
        h = h_new
    return h


# speedup vs baseline: 18.9329x; 18.9329x over previous
import functools

import jax
import jax.numpy as jnp
from jax import lax
from jax.experimental import pallas as pl
from jax.experimental.pallas import tpu as pltpu
from jax.experimental.pallas import tpu_sc as plsc

_N = 10000
_E = 160000
_KERNEL = 25
_DIM = 3
_S = 8
_CHANNELS = [1, 8, 16, 32, 32, 32, 32, 32, 32, 32, 16, 8, 1]

_NC, _NS, _LANES = 2, 16, 16
_NW = _NC * _NS
_EPC = _E // _NW
_CH = 1000
_NCHUNK = _EPC // _CH
_SB = 125
_NSB = _CH // _SB
_B = 2
_NBATCH = _CH // _B
_NP = 10240
_NPS = _NP // _NS


def _pad16(c):
    return ((c + 15) // 16) * 16


def _basis_body(px_ref, py_ref, pz_ref, basis_ref, wi_ref):
    ps = [px_ref[...], py_ref[...], pz_ref[...]]
    v = [p * (_KERNEL - 1) for p in ps]
    lo = [jnp.floor(t) for t in v]
    frac = [t - l for t, l in zip(v, lo)]
    lo_i = [l.astype(jnp.int32) for l in lo]
    for s in range(_S):
        b = jnp.ones_like(ps[0])
        wi = jnp.zeros(ps[0].shape, jnp.int32)
        off = 1
        for d in range(_DIM):
            km = (s >> d) & 1
            b = b * (frac[d] if km == 1 else (1.0 - frac[d]))
            wd = jnp.clip(lo_i[d] + km, 0, _KERNEL - 1)
            wi = wi + wd * off
            off *= _KERNEL
        basis_ref[s, :, :] = b
        wi_ref[s, :, :] = wi


def _spline_basis_pallas(pseudo):
    r, c = 1250, 128
    px = pseudo[:, 0].reshape(r, c)
    py = pseudo[:, 1].reshape(r, c)
    pz = pseudo[:, 2].reshape(r, c)
    basis8, wi8 = pl.pallas_call(
        _basis_body,
        out_shape=(
            jax.ShapeDtypeStruct((_S, r, c), jnp.float32),
            jax.ShapeDtypeStruct((_S, r, c), jnp.int32),
        ),
    )(px, py, pz)
    basis = jnp.moveaxis(basis8.reshape(_S, _E), 0, 1)
    wi = jnp.moveaxis(wi8.reshape(_S, _E), 0, 1)
    return basis, wi


def _cnt_body(dst_hbm, out_hbm, dstb, onesb, zb, shared):
    cid = lax.axis_index("c")
    sid = lax.axis_index("s")
    wid = sid * _NC + cid
    ov = jnp.ones((_LANES,), jnp.float32)
    zv = jnp.zeros((_LANES,), jnp.float32)

    def fill(r, _):
        onesb[r, pl.ds(0, _LANES)] = ov
        return 0

    lax.fori_loop(0, _SB, fill, 0)

    def zfill(r, _):
        zb[r, pl.ds(0, _LANES)] = zv
        return 0

    lax.fori_loop(0, _NPS, zfill, 0)
    pltpu.sync_copy(zb.at[pl.ds(0, _NPS)], shared.at[pl.ds(sid * _NPS, _NPS)])
    plsc.subcore_barrier()

    def chunk_body(ck, _):
        gid = wid * _NCHUNK + ck
        pltpu.sync_copy(dst_hbm.at[pl.ds(gid * _NSB, _NSB)], dstb)
        for j in range(_NSB):
            pltpu.sync_copy(onesb, shared.at[dstb.at[j]], add=True)
        return 0

    lax.fori_loop(0, _NCHUNK, chunk_body, 0)
    plsc.subcore_barrier()
    pltpu.sync_copy(shared.at[pl.ds(sid * _NPS, _NPS)],
                    out_hbm.at[cid, pl.ds(sid * _NPS, _NPS)])


def _cnt_kernel(dst2):
    mesh = plsc.VectorSubcoreMesh(core_axis_name="c", subcore_axis_name="s")
    return pl.kernel(
        _cnt_body,
        mesh=mesh,
        out_type=jax.ShapeDtypeStruct((_NC, _NP, _LANES), jnp.float32),
        compiler_params=pltpu.CompilerParams(use_tc_tiling_on_sc=False),
        scratch_types=[
            pltpu.VMEM((_NSB, _SB), jnp.int32),
            pltpu.VMEM((_SB, _LANES), jnp.float32),
            pltpu.VMEM((_NPS, _LANES), jnp.float32),
            pltpu.VMEM_SHARED((_NP, _LANES), jnp.float32),
        ],
    )(dst2)


def _make_msg_body(cin, cin_p, cout_p, barrier_scatter=False, vmem_idx=False):
    nq = cout_p // _LANES
    nxv = cin_p // _LANES

    def body(h_hbm, w_hbm, src_hbm, dst_hbm, bas_hbm, wi_hbm, out_hbm,
             srcb, dstb, basb, wib, xb, msgb, wbuf, shared, semx, semw):
        cid = lax.axis_index("c")
        sid = lax.axis_index("s")
        wid = sid * _NC + cid
        zv = jnp.zeros((_LANES,), jnp.float32)

        def zb(r, _):
            for q in range(nq):
                msgb[r, pl.ds(q * _LANES, _LANES)] = zv
            return 0

        lax.fori_loop(0, _NPS, zb, 0)
        pltpu.sync_copy(msgb.at[pl.ds(0, _NPS)],
                        shared.at[pl.ds(sid * _NPS, _NPS)])
        plsc.subcore_barrier()

        def chunk_body(ck, _):
            gid = wid * _NCHUNK + ck
            pltpu.sync_copy(src_hbm.at[pl.ds(gid * _NSB, _NSB)], srcb)
            pltpu.sync_copy(dst_hbm.at[pl.ds(gid * _NSB, _NSB)], dstb)
            pltpu.sync_copy(bas_hbm.at[pl.ds(gid * (_CH * _S), _CH * _S)], basb)
            pltpu.sync_copy(wi_hbm.at[pl.ds(gid * (_CH * _S), _CH * _S)], wib)
            for j in range(_NSB):
                pltpu.async_copy(h_hbm.at[srcb.at[j]],
                                 xb.at[pl.ds(j * _SB, _SB)], semx).wait()

            def batch_body(b, _):
                if vmem_idx:
                    idx_ref = wib.at[pl.ds(b * (_B * _S), _B * _S)]
                    pltpu.async_copy(w_hbm.at[idx_ref], wbuf, semw).wait()
                else:
                    idx = wib[pl.ds(b * (_B * _S), _B * _S)]
                    pltpu.async_copy(w_hbm.at[idx], wbuf, semw).wait()
                bv = basb[pl.ds(b * (_B * _S), _B * _S)]
                for e in range(_B):
                    row = b * _B + e
                    xvecs = [xb[row, pl.ds(q * _LANES, _LANES)]
                             for q in range(nxv)]
                    xs = [xvecs[i // _LANES][i % _LANES] for i in range(cin)]
                    accs = [jnp.zeros((_LANES,), jnp.float32)
                            for _ in range(nq)]
                    for s in range(_S):
                        cb = bv[e * _S + s]
                        wrow = e * _S + s
                        sacc = [jnp.zeros((_LANES,), jnp.float32)
                                for _ in range(nq)]
                        for i in range(cin):
                            for q in range(nq):
                                w = wbuf[wrow,
                                         pl.ds(i * cout_p + q * _LANES,
                                               _LANES)]
                                sacc[q] = sacc[q] + xs[i] * w
                        for q in range(nq):
                            accs[q] = accs[q] + cb * sacc[q]
                    for q in range(nq):
                        msgb[row, pl.ds(q * _LANES, _LANES)] = accs[q]
                return 0

            lax.fori_loop(0, _NBATCH, batch_body, 0)
            if barrier_scatter:
                plsc.subcore_barrier()
            for j in range(_NSB):
                pltpu.sync_copy(msgb.at[pl.ds(j * _SB, _SB)],
                                shared.at[dstb.at[j]], add=True)
            return 0

        lax.fori_loop(0, _NCHUNK, chunk_body, 0)
        plsc.subcore_barrier()
        pltpu.sync_copy(shared.at[pl.ds(sid * _NPS, _NPS)],
                        out_hbm.at[cid, pl.ds(sid * _NPS, _NPS)])

    return body


@functools.lru_cache(maxsize=None)
def _msg_kernel(cin, cin_p, cout_p, barrier_scatter=False, vmem_idx=False):
    mesh = plsc.VectorSubcoreMesh(core_axis_name="c", subcore_axis_name="s")
    return pl.kernel(
        _make_msg_body(cin, cin_p, cout_p, barrier_scatter, vmem_idx),
        mesh=mesh,
        out_type=jax.ShapeDtypeStruct((_NC, _NP, cout_p), jnp.float32),
        compiler_params=pltpu.CompilerParams(use_tc_tiling_on_sc=False),
        scratch_types=[
            pltpu.VMEM((_NSB, _SB), jnp.int32),
            pltpu.VMEM((_NSB, _SB), jnp.int32),
            pltpu.VMEM((_CH * _S,), jnp.float32),
            pltpu.VMEM((_CH * _S,), jnp.int32),
            pltpu.VMEM((_CH, cin_p), jnp.float32),
            pltpu.VMEM((_CH, cout_p), jnp.float32),
            pltpu.VMEM((_B * _S, cin * cout_p), jnp.float32),
            pltpu.VMEM_SHARED((_NP, cout_p), jnp.float32),
            pltpu.SemaphoreType.DMA,
            pltpu.SemaphoreType.DMA,
        ],
    )


def _make_dense_body(do_bn):
    def body(p0_ref, p1_ref, c0_ref, c1_ref, h_ref, root_ref, bias_ref,
             gamma_ref, beta_ref, out_ref):
        summed = p0_ref[...] + p1_ref[...]
        cnt = c0_ref[...][:, 0:1] + c1_ref[...][:, 0:1]
        out = summed / jnp.maximum(cnt, 1.0) + h_ref[...] + bias_ref[...]
        if do_bn:
            xn = jnp.minimum(out, 0.0)
            poly = xn * (1.0 + xn * (0.5 + xn * (1.0 / 6.0 + xn * (1.0 / 24.0 + xn * (1.0 / 120.0 + xn / 720.0)))))
            em1 = jnp.where(xn < -0.125, jnp.exp(xn) - 1.0, poly)
            out = jnp.where(out > 0.0, out, em1)
            c = out.shape[1]
            t = out.reshape(100, 100, c)
            mu = jnp.mean(jnp.mean(t, axis=1), axis=0)[None, :]
            d2 = (out - mu) ** 2
            t2 = d2.reshape(100, 100, c)
            var = jnp.mean(jnp.mean(t2, axis=1), axis=0)[None, :]
            out = (out - mu) * lax.rsqrt(var + 1e-5) * gamma_ref[...] \
                + beta_ref[...]
        out_ref[...] = out
    return body


def _dense_call(p0, p1, c0, c1, h, root, bias, gamma, beta, do_bn):
    n, cout_p = p0.shape
    return pl.pallas_call(
        _make_dense_body(do_bn),
        out_shape=jax.ShapeDtypeStruct((n, cout_p), jnp.float32),
    )(p0, p1, c0, c1, h, root, bias, gamma, beta)


def _make_k1_body(exact_dot, do_elu):
    def _k1_body(p0_ref, p1_ref, c0_ref, c1_ref, h_ref, root_ref, bias_ref,
                 out_ref):
        summed = p0_ref[...] + p1_ref[...]
        cnt = c0_ref[...][:, 0:1] + c1_ref[...][:, 0:1]
        hv = h_ref[...]
        rv = root_ref[...]

        def _mm(a, b):
            r = a[:, 0:1] * b[0:1, :]
            for i in range(1, a.shape[1]):
                r = r + a[:, i:i + 1] * b[i:i + 1, :]
            return r

        if exact_dot:
            rt = _mm(hv, rv)
        else:
            ah = hv.astype(jnp.bfloat16).astype(jnp.float32)
            bh = rv.astype(jnp.bfloat16).astype(jnp.float32)
            rt = _mm(ah, bh)
        out = summed / jnp.maximum(cnt, 1.0) + rt + bias_ref[...]
        if do_elu:
            xn = jnp.minimum(out, 0.0)
            poly = xn * (1.0 + xn * (0.5 + xn * (1.0 / 6.0 + xn * (
                1.0 / 24.0 + xn * (1.0 / 120.0 + xn / 720.0)))))
            em1 = jnp.where(xn < -0.125, jnp.exp(xn) - 1.0, poly)
            out = jnp.where(out > 0.0, out, em1)
        out_ref[...] = out
    return _k1_body


def _bn_body(x_ref, gamma_ref, beta_ref, out_ref):
    xv = x_ref[...]
    mu = jnp.mean(xv, axis=0, keepdims=True)
    d = xv - mu
    var = jnp.mean(d * d, axis=0, keepdims=True)
    out_ref[...] = d * lax.rsqrt(var + 1e-5) * gamma_ref[...] + beta_ref[...]


def _bn_call(x, gamma, beta):
    return pl.pallas_call(
        _bn_body,
        out_shape=jax.ShapeDtypeStruct(x.shape, jnp.float32),
    )(x, gamma, beta)


def _k1_call(p0, p1, c0, c1, h, root, bias, exact_dot=False, do_elu=False):
    n, cout_p = p0.shape
    cin_p = h.shape[1]
    nb = 10
    blk = n // nb
    return pl.pallas_call(
        _make_k1_body(exact_dot, do_elu),
        grid=(nb,),
        in_specs=[
            pl.BlockSpec((blk, cout_p), lambda j: (j, 0)),
            pl.BlockSpec((blk, cout_p), lambda j: (j, 0)),
            pl.BlockSpec((blk, _LANES), lambda j: (j, 0)),
            pl.BlockSpec((blk, _LANES), lambda j: (j, 0)),
            pl.BlockSpec((blk, cin_p), lambda j: (j, 0)),
            pl.BlockSpec((cin_p, cout_p), lambda j: (0, 0)),
            pl.BlockSpec((1, cout_p), lambda j: (0, 0)),
        ],
        out_specs=pl.BlockSpec((blk, cout_p), lambda j: (j, 0)),
        out_shape=jax.ShapeDtypeStruct((n, cout_p), jnp.float32),
    )(p0, p1, c0, c1, h, root, bias)


def _kernel_full(x, edge_index, pseudo, params):
    src = edge_index[0].astype(jnp.int32)
    dst = edge_index[1].astype(jnp.int32)

    basis, wi = _spline_basis_pallas(pseudo)
    bas2 = basis.reshape(_E * _S)
    wi2 = wi.reshape(_E * _S)
    src2 = src.reshape(_E // _SB, _SB)
    dst2 = dst.reshape(_E // _SB, _SB)

    cnt = _cnt_kernel(dst2)
    c0, c1 = cnt[0, :_N], cnt[1, :_N]

    h = jnp.pad(x, ((0, 0), (0, 15)))
    for i in range(1, 13):
        cin, cout = _CHANNELS[i - 1], _CHANNELS[i]
        cin_p, cout_p = _pad16(cin), _pad16(cout)
        W = params[f"W{i}"]
        W_p = jnp.pad(W, ((0, 0), (0, 0), (0, cout_p - cout)))
        W_p = W_p.reshape(W.shape[0], cin * cout_p)
        root_p = jnp.pad(params[f"root{i}"],
                         ((0, cin_p - cin), (0, cout_p - cout)))
        bias_p = jnp.pad(params[f"bias{i}"], (0, cout_p - cout))[None, :]
        if i < 12:
            gamma_p = jnp.pad(params[f"gamma{i}"], (0, cout_p - cout))[None, :]
            beta_p = jnp.pad(params[f"beta{i}"], (0, cout_p - cout))[None, :]
        else:
            gamma_p = jnp.zeros((1, cout_p), jnp.float32)
            beta_p = jnp.zeros((1, cout_p), jnp.float32)

        partial = _msg_kernel(cin, cin_p, cout_p)(
            h, W_p, src2, dst2, bas2, wi2)
        h = _dense_call(partial[0, :_N], partial[1, :_N], c0, c1, h,
                        root_p, bias_p, gamma_p, beta_p, do_bn=(i < 12))

    return h[:, 0:1]


def kernel(x, edge_index, pseudo, params):
    src = edge_index[0].astype(jnp.int32)
    dst = edge_index[1].astype(jnp.int32)
    basis, wi = _spline_basis_pallas(pseudo)
    bas2 = basis.reshape(_E * _S)
    wi2 = wi.reshape(_E * _S)
    src2 = src.reshape(_E // _SB, _SB)
    dst2 = dst.reshape(_E // _SB, _SB)
    cnt = _cnt_kernel(dst2)
    c0, c1 = cnt[0, :_N], cnt[1, :_N]
    h = x
    hp = jnp.pad(x, ((0, 0), (0, 15)))
    for i in range(1, 13):
        cin, cout = _CHANNELS[i - 1], _CHANNELS[i]
        cin_p, cout_p = _pad16(cin), _pad16(cout)
        W = params[f"W{i}"]
        W_p = jnp.pad(W, ((0, 0), (0, 0), (0, cout_p - cout)))
        W_p = W_p.reshape(W.shape[0], cin * cout_p)
        root_p = jnp.pad(params[f"root{i}"],
                         ((0, cin_p - cin), (0, cout_p - cout)))
        bias_p = jnp.pad(params[f"bias{i}"], (0, cout_p - cout))[None, :]
        partial = _msg_kernel(cin, cin_p, cout_p)(hp, W_p, src2, dst2, bas2, wi2)
        hd = _k1_call(partial[0, :_N], partial[1, :_N], c0, c1, hp,
                      root_p, bias_p, exact_dot=(cin == 1), do_elu=(i < 12))
        if i < 12:
            gamma_p = jnp.pad(params[f"gamma{i}"], (0, cout_p - cout))[None, :]
            beta_p = jnp.pad(params[f"beta{i}"], (0, cout_p - cout))[None, :]
            hp = _bn_call(hd, gamma_p, beta_p)
        else:
            hp = hd
    return hp[:, 0:1]


# speedup vs baseline: 18.9870x; 1.0029x over previous
import functools

import jax
import jax.numpy as jnp
from jax import lax
from jax.experimental import pallas as pl
from jax.experimental.pallas import tpu as pltpu
from jax.experimental.pallas import tpu_sc as plsc

_N = 10000
_E = 160000
_KERNEL = 25
_DIM = 3
_S = 8
_CHANNELS = [1, 8, 16, 32, 32, 32, 32, 32, 32, 32, 16, 8, 1]

_NC, _NS, _LANES = 2, 16, 16
_NW = _NC * _NS
_EPC = _E // _NW
_CH = 1000
_NCHUNK = _EPC // _CH
_SB = 125
_NSB = _CH // _SB
_B = 2
_NBATCH = _CH // _B
_NP = 10240
_NPS = _NP // _NS


def _pad16(c):
    return ((c + 15) // 16) * 16


def _basis_body(px_ref, py_ref, pz_ref, basis_ref, wi_ref):
    ps = [px_ref[...], py_ref[...], pz_ref[...]]
    v = [p * (_KERNEL - 1) for p in ps]
    lo = [jnp.floor(t) for t in v]
    frac = [t - l for t, l in zip(v, lo)]
    lo_i = [l.astype(jnp.int32) for l in lo]
    for s in range(_S):
        b = jnp.ones_like(ps[0])
        wi = jnp.zeros(ps[0].shape, jnp.int32)
        off = 1
        for d in range(_DIM):
            km = (s >> d) & 1
            b = b * (frac[d] if km == 1 else (1.0 - frac[d]))
            wd = jnp.clip(lo_i[d] + km, 0, _KERNEL - 1)
            wi = wi + wd * off
            off *= _KERNEL
        basis_ref[s, :, :] = b
        wi_ref[s, :, :] = wi


def _spline_basis_pallas(pseudo):
    r, c = 1250, 128
    px = pseudo[:, 0].reshape(r, c)
    py = pseudo[:, 1].reshape(r, c)
    pz = pseudo[:, 2].reshape(r, c)
    basis8, wi8 = pl.pallas_call(
        _basis_body,
        out_shape=(
            jax.ShapeDtypeStruct((_S, r, c), jnp.float32),
            jax.ShapeDtypeStruct((_S, r, c), jnp.int32),
        ),
    )(px, py, pz)
    basis = jnp.moveaxis(basis8.reshape(_S, _E), 0, 1)
    wi = jnp.moveaxis(wi8.reshape(_S, _E), 0, 1)
    return basis, wi


def _cnt_body(dst_hbm, out_hbm, dstb, onesb, zb, shared):
    cid = lax.axis_index("c")
    sid = lax.axis_index("s")
    wid = sid * _NC + cid
    ov = jnp.ones((_LANES,), jnp.float32)
    zv = jnp.zeros((_LANES,), jnp.float32)

    def fill(r, _):
        onesb[r, pl.ds(0, _LANES)] = ov
        return 0

    lax.fori_loop(0, _SB, fill, 0)

    def zfill(r, _):
        zb[r, pl.ds(0, _LANES)] = zv
        return 0

    lax.fori_loop(0, _NPS, zfill, 0)
    pltpu.sync_copy(zb.at[pl.ds(0, _NPS)], shared.at[pl.ds(sid * _NPS, _NPS)])
    plsc.subcore_barrier()

    def chunk_body(ck, _):
        gid = wid * _NCHUNK + ck
        pltpu.sync_copy(dst_hbm.at[pl.ds(gid * _NSB, _NSB)], dstb)
        for j in range(_NSB):
            pltpu.sync_copy(onesb, shared.at[dstb.at[j]], add=True)
        return 0

    lax.fori_loop(0, _NCHUNK, chunk_body, 0)
    plsc.subcore_barrier()
    pltpu.sync_copy(shared.at[pl.ds(sid * _NPS, _NPS)],
                    out_hbm.at[cid, pl.ds(sid * _NPS, _NPS)])


def _cnt_kernel(dst2):
    mesh = plsc.VectorSubcoreMesh(core_axis_name="c", subcore_axis_name="s")
    return pl.kernel(
        _cnt_body,
        mesh=mesh,
        out_type=jax.ShapeDtypeStruct((_NC, _NP, _LANES), jnp.float32),
        compiler_params=pltpu.CompilerParams(use_tc_tiling_on_sc=False),
        scratch_types=[
            pltpu.VMEM((_NSB, _SB), jnp.int32),
            pltpu.VMEM((_SB, _LANES), jnp.float32),
            pltpu.VMEM((_NPS, _LANES), jnp.float32),
            pltpu.VMEM_SHARED((_NP, _LANES), jnp.float32),
        ],
    )(dst2)


def _make_msg_body(cin, cin_p, cout_p, barrier_scatter=False, vmem_idx=False):
    nq = cout_p // _LANES
    nxv = cin_p // _LANES

    def body(h_hbm, w_hbm, src_hbm, dst_hbm, bas_hbm, wi_hbm, out_hbm,
             srcb, dstb, basb, wib, xb, msgb, wbuf, shared, semx, semw):
        cid = lax.axis_index("c")
        sid = lax.axis_index("s")
        wid = sid * _NC + cid
        zv = jnp.zeros((_LANES,), jnp.float32)

        def zb(r, _):
            for q in range(nq):
                msgb[r, pl.ds(q * _LANES, _LANES)] = zv
            return 0

        lax.fori_loop(0, _NPS, zb, 0)
        pltpu.sync_copy(msgb.at[pl.ds(0, _NPS)],
                        shared.at[pl.ds(sid * _NPS, _NPS)])
        plsc.subcore_barrier()

        def chunk_body(ck, _):
            gid = wid * _NCHUNK + ck
            pltpu.sync_copy(src_hbm.at[pl.ds(gid * _NSB, _NSB)], srcb)
            pltpu.sync_copy(dst_hbm.at[pl.ds(gid * _NSB, _NSB)], dstb)
            pltpu.sync_copy(bas_hbm.at[pl.ds(gid * (_CH * _S), _CH * _S)], basb)
            pltpu.sync_copy(wi_hbm.at[pl.ds(gid * (_CH * _S), _CH * _S)], wib)
            for j in range(_NSB):
                pltpu.async_copy(h_hbm.at[srcb.at[j]],
                                 xb.at[pl.ds(j * _SB, _SB)], semx).wait()

            def batch_body(b, _):
                if vmem_idx:
                    idx_ref = wib.at[pl.ds(b * (_B * _S), _B * _S)]
                    pltpu.async_copy(w_hbm.at[idx_ref], wbuf, semw).wait()
                else:
                    idx = wib[pl.ds(b * (_B * _S), _B * _S)]
                    pltpu.async_copy(w_hbm.at[idx], wbuf, semw).wait()
                bv = basb[pl.ds(b * (_B * _S), _B * _S)]
                for e in range(_B):
                    row = b * _B + e
                    xvecs = [xb[row, pl.ds(q * _LANES, _LANES)]
                             for q in range(nxv)]
                    xs = [xvecs[i // _LANES][i % _LANES] for i in range(cin)]
                    accs = [jnp.zeros((_LANES,), jnp.float32)
                            for _ in range(nq)]
                    for s in range(_S):
                        cb = bv[e * _S + s]
                        wrow = e * _S + s
                        sacc = [jnp.zeros((_LANES,), jnp.float32)
                                for _ in range(nq)]
                        for i in range(cin):
                            for q in range(nq):
                                w = wbuf[wrow,
                                         pl.ds(i * cout_p + q * _LANES,
                                               _LANES)]
                                sacc[q] = sacc[q] + xs[i] * w
                        for q in range(nq):
                            accs[q] = accs[q] + cb * sacc[q]
                    for q in range(nq):
                        msgb[row, pl.ds(q * _LANES, _LANES)] = accs[q]
                return 0

            lax.fori_loop(0, _NBATCH, batch_body, 0)
            if barrier_scatter:
                plsc.subcore_barrier()
            for j in range(_NSB):
                pltpu.sync_copy(msgb.at[pl.ds(j * _SB, _SB)],
                                shared.at[dstb.at[j]], add=True)
            return 0

        lax.fori_loop(0, _NCHUNK, chunk_body, 0)
        plsc.subcore_barrier()
        pltpu.sync_copy(shared.at[pl.ds(sid * _NPS, _NPS)],
                        out_hbm.at[cid, pl.ds(sid * _NPS, _NPS)])

    return body


@functools.lru_cache(maxsize=None)
def _msg_kernel(cin, cin_p, cout_p, barrier_scatter=False, vmem_idx=False):
    mesh = plsc.VectorSubcoreMesh(core_axis_name="c", subcore_axis_name="s")
    return pl.kernel(
        _make_msg_body(cin, cin_p, cout_p, barrier_scatter, vmem_idx),
        mesh=mesh,
        out_type=jax.ShapeDtypeStruct((_NC, _NP, cout_p), jnp.float32),
        compiler_params=pltpu.CompilerParams(use_tc_tiling_on_sc=False),
        scratch_types=[
            pltpu.VMEM((_NSB, _SB), jnp.int32),
            pltpu.VMEM((_NSB, _SB), jnp.int32),
            pltpu.VMEM((_CH * _S,), jnp.float32),
            pltpu.VMEM((_CH * _S,), jnp.int32),
            pltpu.VMEM((_CH, cin_p), jnp.float32),
            pltpu.VMEM((_CH, cout_p), jnp.float32),
            pltpu.VMEM((_B * _S, cin * cout_p), jnp.float32),
            pltpu.VMEM_SHARED((_NP, cout_p), jnp.float32),
            pltpu.SemaphoreType.DMA,
            pltpu.SemaphoreType.DMA,
        ],
    )


def _make_k1_body(exact_dot, do_elu):
    def _k1_body(p0_ref, p1_ref, c0_ref, c1_ref, h_ref, root_ref, bias_ref,
                 out_ref):
        summed = p0_ref[...] + p1_ref[...]
        cnt = c0_ref[...][:, 0:1] + c1_ref[...][:, 0:1]
        hv = h_ref[...]
        rv = root_ref[...]

        def _mm(a, b):
            r = a[:, 0:1] * b[0:1, :]
            for i in range(1, a.shape[1]):
                r = r + a[:, i:i + 1] * b[i:i + 1, :]
            return r

        if exact_dot:
            rt = _mm(hv, rv)
        else:
            ah = hv.astype(jnp.bfloat16).astype(jnp.float32)
            bh = rv.astype(jnp.bfloat16).astype(jnp.float32)
            rt = _mm(ah, bh)
        out = summed / jnp.maximum(cnt, 1.0) + rt + bias_ref[...]
        if do_elu:
            xn = jnp.minimum(out, 0.0)
            poly = xn * (1.0 + xn * (0.5 + xn * (1.0 / 6.0 + xn * (
                1.0 / 24.0 + xn * (1.0 / 120.0 + xn / 720.0)))))
            em1 = jnp.where(xn < -0.125, jnp.exp(xn) - 1.0, poly)
            out = jnp.where(out > 0.0, out, em1)
        out_ref[...] = out
    return _k1_body


def _bn_body(x_ref, gamma_ref, beta_ref, out_ref):
    xv = x_ref[...]
    mu = jnp.mean(xv, axis=0, keepdims=True)
    d = xv - mu
    var = jnp.mean(d * d, axis=0, keepdims=True)
    out_ref[...] = d * lax.rsqrt(var + 1e-5) * gamma_ref[...] + beta_ref[...]


def _bn_call(x, gamma, beta):
    return pl.pallas_call(
        _bn_body,
        out_shape=jax.ShapeDtypeStruct(x.shape, jnp.float32),
    )(x, gamma, beta)


def _k1_call(p0, p1, c0, c1, h, root, bias, exact_dot=False, do_elu=False):
    n, cout_p = p0.shape
    cin_p = h.shape[1]
    nb = 10
    blk = n // nb
    return pl.pallas_call(
        _make_k1_body(exact_dot, do_elu),
        grid=(nb,),
        in_specs=[
            pl.BlockSpec((blk, cout_p), lambda j: (j, 0)),
            pl.BlockSpec((blk, cout_p), lambda j: (j, 0)),
            pl.BlockSpec((blk, _LANES), lambda j: (j, 0)),
            pl.BlockSpec((blk, _LANES), lambda j: (j, 0)),
            pl.BlockSpec((blk, cin_p), lambda j: (j, 0)),
            pl.BlockSpec((cin_p, cout_p), lambda j: (0, 0)),
            pl.BlockSpec((1, cout_p), lambda j: (0, 0)),
        ],
        out_specs=pl.BlockSpec((blk, cout_p), lambda j: (j, 0)),
        out_shape=jax.ShapeDtypeStruct((n, cout_p), jnp.float32),
    )(p0, p1, c0, c1, h, root, bias)


def kernel(x, edge_index, pseudo, params):
    src = edge_index[0].astype(jnp.int32)
    dst = edge_index[1].astype(jnp.int32)
    basis, wi = _spline_basis_pallas(pseudo)
    bas2 = basis.reshape(_E * _S)
    wi2 = wi.reshape(_E * _S)
    src2 = src.reshape(_E // _SB, _SB)
    dst2 = dst.reshape(_E // _SB, _SB)
    cnt = _cnt_kernel(dst2)
    c0, c1 = cnt[0, :_N], cnt[1, :_N]
    h = x
    hp = jnp.pad(x, ((0, 0), (0, 15)))
    for i in range(1, 13):
        cin, cout = _CHANNELS[i - 1], _CHANNELS[i]
        cin_p, cout_p = _pad16(cin), _pad16(cout)
        W = params[f"W{i}"]
        W_p = jnp.pad(W, ((0, 0), (0, 0), (0, cout_p - cout)))
        W_p = W_p.reshape(W.shape[0], cin * cout_p)
        root_p = jnp.pad(params[f"root{i}"],
                         ((0, cin_p - cin), (0, cout_p - cout)))
        bias_p = jnp.pad(params[f"bias{i}"], (0, cout_p - cout))[None, :]
        partial = _msg_kernel(cin, cin_p, cout_p)(hp, W_p, src2, dst2, bas2, wi2)
        hd = _k1_call(partial[0, :_N], partial[1, :_N], c0, c1, hp,
                      root_p, bias_p, exact_dot=(cin == 1), do_elu=(i < 12))
        if i < 12:
            gamma_p = jnp.pad(params[f"gamma{i}"], (0, cout_p - cout))[None, :]
            beta_p = jnp.pad(params[f"beta{i}"], (0, cout_p - cout))[None, :]
            hp = _bn_call(hd, gamma_p, beta_p)
        else:
            hp = hd
    return hp[:, 0:1]
